# baseline (speedup 1.0000x reference)
import jax
import jax.numpy as jnp
from jax.experimental import pallas as pl
from jax.experimental.pallas import tpu as pltpu


def _body(x_ref, a0_ref, a1_ref, w_ref, b_ref, o_ref):
    dn = (((1,), (0,)), ((), ()))
    dnt = (((0,), (0,)), ((), ()))
    f32 = jnp.float32
    bf16 = jnp.bfloat16
    N, wt = x_ref.shape
    n_groups, wc, _ = w_ref.shape
    n_sub = wt // wc

    xb = x_ref[...]
    feats = [xb]
    for a_ref in (a0_ref, a1_ref):
        a = a_ref[...]
        h1 = jax.lax.dot_general(a, xb, dnt, preferred_element_type=f32)
        h1 = h1.astype(bf16)
        h2 = jax.lax.dot_general(a, h1, dnt, preferred_element_type=f32)
        feats.append(h1)
        feats.append(h2.astype(bf16))

    for k in range(n_sub):
        acc = jnp.broadcast_to(b_ref[...], (N, wc)).astype(f32)
        for g, f in enumerate(feats):
            acc += jax.lax.dot_general(
                f[:, k * wc:(k + 1) * wc], w_ref[g], dn,
                preferred_element_type=f32)
        o_ref[:, k * wc:(k + 1) * wc] = acc.astype(bf16)


def kernel(x, adjs, W, b):
    B, C, N, T = x.shape
    out_ch, in_ch = W.shape

    nbt = 4
    wc = nbt * C
    wt = 1 * wc
    cols = B * T * C
    grid = cols // wt

    bf16 = jnp.bfloat16
    xt = jnp.transpose(x, (2, 0, 3, 1)).reshape(N, cols).astype(bf16)
    a0 = adjs[0].astype(bf16)
    a1 = adjs[1].astype(bf16)
    eye = jnp.eye(nbt, dtype=W.dtype)
    wbd = jnp.stack(
        [jnp.kron(eye, W[:, g * C:(g + 1) * C].T) for g in range(in_ch // C)]
    ).astype(bf16)
    bt = jnp.tile(b, nbt)[None, :].astype(jnp.float32)

    out2d = pl.pallas_call(
        _body,
        grid=(grid,),
        in_specs=[
            pl.BlockSpec((N, wt), lambda j: (0, j)),
            pl.BlockSpec((N, N), lambda j: (0, 0)),
            pl.BlockSpec((N, N), lambda j: (0, 0)),
            pl.BlockSpec(wbd.shape, lambda j: (0, 0, 0)),
            pl.BlockSpec((1, wc), lambda j: (0, 0)),
        ],
        out_specs=pl.BlockSpec((N, wt), lambda j: (0, j)),
        out_shape=jax.ShapeDtypeStruct((N, cols), bf16),
        compiler_params=pltpu.CompilerParams(
            dimension_semantics=("parallel",),
            allow_input_fusion=[True, True, True, True, True],
        ),
    )(xt, a0, a1, wbd, bt)

    return out2d.reshape(N, B, T, out_ch).transpose(1, 3, 0, 2).astype(
        jnp.float32)


# speedup vs baseline: 1.0695x; 1.0695x over previous
import jax
import jax.numpy as jnp
from jax.experimental import pallas as pl
from jax.experimental.pallas import tpu as pltpu


def _body(x_ref, a0_ref, a1_ref, w_ref, b_ref, o_ref):
    dn = (((1,), (0,)), ((), ()))
    dnt = (((0,), (0,)), ((), ()))
    f32 = jnp.float32
    bf16 = jnp.bfloat16
    N, wt = x_ref.shape
    n_groups, wc, _ = w_ref.shape
    n_sub = wt // wc

    xb = x_ref[...]
    feats = [xb]
    for a_ref in (a0_ref, a1_ref):
        a = a_ref[...]
        h1 = jax.lax.dot_general(a, xb, dnt, preferred_element_type=f32)
        h1 = h1.astype(bf16)
        h2 = jax.lax.dot_general(a, h1, dnt, preferred_element_type=f32)
        feats.append(h1)
        feats.append(h2.astype(bf16))

    for k in range(n_sub):
        acc = jnp.broadcast_to(b_ref[...], (N, wc)).astype(f32)
        for g, f in enumerate(feats):
            acc += jax.lax.dot_general(
                f[:, k * wc:(k + 1) * wc], w_ref[g], dn,
                preferred_element_type=f32)
        o_ref[:, k * wc:(k + 1) * wc] = acc.astype(bf16)


def kernel(x, adjs, W, b):
    B, C, N, T = x.shape
    out_ch, in_ch = W.shape

    nbt = 4
    wc = nbt * C
    wt = 4 * wc
    cols = B * T * C
    grid = cols // wt

    bf16 = jnp.bfloat16
    xt = jnp.transpose(x, (2, 0, 3, 1)).reshape(N, cols).astype(bf16)
    a0 = adjs[0].astype(bf16)
    a1 = adjs[1].astype(bf16)
    eye = jnp.eye(nbt, dtype=W.dtype)
    wbd = jnp.stack(
        [jnp.kron(eye, W[:, g * C:(g + 1) * C].T) for g in range(in_ch // C)]
    ).astype(bf16)
    bt = jnp.tile(b, nbt)[None, :].astype(jnp.float32)

    out2d = pl.pallas_call(
        _body,
        grid=(grid,),
        in_specs=[
            pl.BlockSpec((N, wt), lambda j: (0, j)),
            pl.BlockSpec((N, N), lambda j: (0, 0)),
            pl.BlockSpec((N, N), lambda j: (0, 0)),
            pl.BlockSpec(wbd.shape, lambda j: (0, 0, 0)),
            pl.BlockSpec((1, wc), lambda j: (0, 0)),
        ],
        out_specs=pl.BlockSpec((N, wt), lambda j: (0, j)),
        out_shape=jax.ShapeDtypeStruct((N, cols), bf16),
        compiler_params=pltpu.CompilerParams(
            dimension_semantics=("parallel",),
            allow_input_fusion=[True, True, True, True, True],
        ),
    )(xt, a0, a1, wbd, bt)

    return out2d.reshape(N, B, T, out_ch).transpose(1, 3, 0, 2).astype(
        jnp.float32)
